# baseline (speedup 1.0000x reference)
import functools

import jax
import jax.numpy as jnp
from jax import lax
from jax.experimental import pallas as pl
from jax.experimental.pallas import tpu as pltpu
from jax.experimental.pallas import tpu_sc as plsc

_BN_EPS = 1e-5


def _sc_aggregate(x, edge_index, *, NC, NS, C, K, RPT):
    _, D = x.shape
    Npad = RPT * NS
    EPW = C * K
    E = edge_index.shape[1]
    L = 16
    assert K % L == 0
    WS = (EPW + 2 * 128 - 1) // 128 * 128
    mesh = plsc.VectorSubcoreMesh(core_axis_name="c", subcore_axis_name="s")

    @functools.partial(
        pl.kernel,
        out_type=jax.ShapeDtypeStruct((NC, Npad, D), jnp.float32),
        mesh=mesh,
        scratch_types=[
            pltpu.VMEM((2, WS), jnp.int32),
            pltpu.VMEM((K,), jnp.int32),
            pltpu.VMEM((K,), jnp.int32),
            pltpu.VMEM((K,), jnp.int32),
            pltpu.VMEM((K,), jnp.int32),
            pltpu.VMEM((K, D), jnp.float32),
            pltpu.VMEM((K, D), jnp.float32),
            pltpu.VMEM_SHARED((Npad, D), jnp.float32),
            pltpu.SemaphoreType.DMA,
            pltpu.SemaphoreType.DMA,
        ],
    )
    def agg(x_hbm, ei_hbm, out_hbm,
            win_v, sb0, sb1, db0, db1, rows0, rows1, accum_sh, gsem0, gsem1):
        cid = lax.axis_index("c")
        sid = lax.axis_index("s")
        wid = sid * NC + cid

        begin = wid * EPW
        start = jnp.minimum(begin - begin % 128, E - WS)
        start = pl.multiple_of(start, 128)
        sh = pl.multiple_of(begin - start, L)

        def untangle(row, j, buf):
            for i in range(K // L):
                buf[pl.ds(i * L, L)] = win_v[row,
                                             pl.ds(sh + j * K + i * L, L)]

        def gather(j, rows, sb, gsem):
            untangle(0, j, sb)
            return pltpu.async_copy(x_hbm.at[sb], rows, gsem)

        def gather_wait(rows, gsem):
            pltpu.make_async_copy(x_hbm.at[sb0], rows, gsem).wait()

        def scat(j, rows, db):
            untangle(1, j, db)
            pltpu.sync_copy(rows, accum_sh.at[db], add=True)

        pltpu.sync_copy(ei_hbm.at[:, pl.ds(start, WS)], win_v)
        gather(0, rows0, sb0, gsem0)

        zv = jnp.zeros((16,), jnp.float32)

        def zrow(r, _):
            for c in range(D // 16):
                rows1[r, pl.ds(c * 16, 16)] = zv
            return _

        lax.fori_loop(0, K, zrow, 0, unroll=False)
        full, rem = RPT // K, RPT % K
        for t in range(full):
            pltpu.async_copy(rows1,
                             accum_sh.at[pl.ds(sid * RPT + t * K, K)], gsem1)
        if rem:
            pltpu.async_copy(rows1.at[pl.ds(0, rem)],
                             accum_sh.at[pl.ds(sid * RPT + full * K, rem)],
                             gsem1)
        for t in range(full):
            pltpu.make_async_copy(
                rows1, accum_sh.at[pl.ds(sid * RPT + t * K, K)], gsem1).wait()
        if rem:
            pltpu.make_async_copy(
                rows1.at[pl.ds(0, rem)],
                accum_sh.at[pl.ds(sid * RPT + full * K, rem)], gsem1).wait()
        gather(1, rows1, sb1, gsem1)
        plsc.subcore_barrier()

        def body(i, _):
            a = 2 * i
            gather_wait(rows0, gsem0)
            scat(a, rows0, db0)

            @pl.when(a + 2 < C)
            def _g0():
                gather(a + 2, rows0, sb0, gsem0)

            gather_wait(rows1, gsem1)
            scat(a + 1, rows1, db1)

            @pl.when(a + 3 < C)
            def _g1():
                gather(a + 3, rows1, sb1, gsem1)

            return _

        lax.fori_loop(0, C // 2, body, 0, unroll=2)
        if C % 2:
            gather_wait(rows0, gsem0)
            scat(C - 1, rows0, db0)
        plsc.subcore_barrier()

        pltpu.sync_copy(accum_sh.at[pl.ds(sid * RPT, RPT)],
                        out_hbm.at[cid, pl.ds(sid * RPT, RPT)])

    return agg(x, edge_index)


def _tc_mlp_body(eps_ref, x_ref, p_ref, w1_ref, b1_ref, g1_ref,
                 be1_ref, w2_ref, b2_ref, g2_ref, be2_ref, o_ref):
    n = x_ref.shape[0]
    h = (1.0 + eps_ref[0]) * x_ref[...] + (p_ref[0, :n] + p_ref[1, :n])
    h1 = jnp.dot(h, w1_ref[...], preferred_element_type=jnp.float32)
    h1 = h1 + b1_ref[...]
    m1 = jnp.mean(h1, axis=0, keepdims=True)
    v1 = jnp.mean((h1 - m1) ** 2, axis=0, keepdims=True)
    h1 = g1_ref[...] * (h1 - m1) * lax.rsqrt(v1 + _BN_EPS) + be1_ref[...]
    h1 = jnp.maximum(h1, 0.0)
    h2 = jnp.dot(h1, w2_ref[...], preferred_element_type=jnp.float32)
    h2 = h2 + b2_ref[...]
    m2 = jnp.mean(h2, axis=0, keepdims=True)
    v2 = jnp.mean((h2 - m2) ** 2, axis=0, keepdims=True)
    h2 = g2_ref[...] * (h2 - m2) * lax.rsqrt(v2 + _BN_EPS) + be2_ref[...]
    o_ref[...] = jnp.maximum(h2, 0.0)


def _tc_mlp(x, partials, W1t, b1, g1, be1, W2t, b2, g2, be2, eps):
    N, D = x.shape
    row = lambda a: a.reshape(1, D)
    return pl.pallas_call(
        _tc_mlp_body,
        out_shape=jax.ShapeDtypeStruct((N, D), jnp.float32),
        in_specs=[pl.BlockSpec(memory_space=pltpu.SMEM)]
        + [pl.BlockSpec(memory_space=pltpu.VMEM)] * 10,
    )(eps.reshape(1), x, partials, W1t, row(b1), row(g1), row(be1),
      W2t, row(b2), row(g2), row(be2))


def kernel(x, edge_index, batch, W1, b1, g1, be1, W2, b2, g2, be2, eps):
    N, D = x.shape
    E = edge_index.shape[1]
    info = plsc.get_sparse_core_info()
    NC, NS = info.num_cores, info.num_subcores
    NW = NC * NS
    EPW = E // NW
    assert E % NW == 0
    K = next(k for k in range(128, 0, -8) if EPW % k == 0)
    C = EPW // K
    assert C >= 2
    RPT = (((N + NS - 1) // NS) + 7) // 8 * 8
    Npad = RPT * NS

    partials = _sc_aggregate(x, edge_index,
                             NC=NC, NS=NS, C=C, K=K, RPT=RPT)
    return _tc_mlp(x, partials, W1.T, b1, g1, be1,
                   W2.T, b2, g2, be2, eps)


# speedup vs baseline: 1.0012x; 1.0012x over previous
import functools

import jax
import jax.numpy as jnp
from jax import lax
from jax.experimental import pallas as pl
from jax.experimental.pallas import tpu as pltpu
from jax.experimental.pallas import tpu_sc as plsc

_BN_EPS = 1e-5


def _sc_aggregate(x, edge_index, *, NC, NS, C, K, RPT):
    _, D = x.shape
    Npad = RPT * NS
    EPW = C * K
    E = edge_index.shape[1]
    L = 16
    assert K % L == 0
    WS = (EPW + 2 * 128 - 1) // 128 * 128
    mesh = plsc.VectorSubcoreMesh(core_axis_name="c", subcore_axis_name="s")

    @functools.partial(
        pl.kernel,
        out_type=jax.ShapeDtypeStruct((NC, Npad, D), jnp.float32),
        mesh=mesh,
        scratch_types=[
            pltpu.VMEM((2, WS), jnp.int32),
            pltpu.VMEM((K,), jnp.int32),
            pltpu.VMEM((K,), jnp.int32),
            pltpu.VMEM((K,), jnp.int32),
            pltpu.VMEM((K,), jnp.int32),
            pltpu.VMEM((K, D), jnp.float32),
            pltpu.VMEM((K, D), jnp.float32),
            pltpu.VMEM_SHARED((Npad, D), jnp.float32),
            pltpu.SemaphoreType.DMA,
            pltpu.SemaphoreType.DMA,
        ],
    )
    def agg(x_hbm, ei_hbm, out_hbm,
            win_v, sb0, sb1, db0, db1, rows0, rows1, accum_sh, gsem0, gsem1):
        cid = lax.axis_index("c")
        sid = lax.axis_index("s")
        wid = sid * NC + cid

        begin = wid * EPW
        start = jnp.minimum(begin - begin % 128, E - WS)
        start = pl.multiple_of(start, 128)
        sh = pl.multiple_of(begin - start, L)

        def untangle(row, j, buf):
            for i in range(K // L):
                buf[pl.ds(i * L, L)] = win_v[row,
                                             pl.ds(sh + j * K + i * L, L)]

        def gather(j, rows, sb, gsem):
            untangle(0, j, sb)
            return pltpu.async_copy(x_hbm.at[sb], rows, gsem)

        def gather_wait(rows, gsem):
            pltpu.make_async_copy(x_hbm.at[sb0], rows, gsem).wait()

        def scat(j, rows, db):
            untangle(1, j, db)
            pltpu.sync_copy(rows, accum_sh.at[db], add=True)

        pltpu.sync_copy(ei_hbm.at[:, pl.ds(start, WS)], win_v)
        gather(0, rows0, sb0, gsem0)

        zv = jnp.zeros((16,), jnp.float32)

        def zrow(r, _):
            for c in range(D // 16):
                rows1[r, pl.ds(c * 16, 16)] = zv
            return _

        lax.fori_loop(0, K, zrow, 0, unroll=False)
        full, rem = RPT // K, RPT % K
        for t in range(full):
            pltpu.async_copy(rows1,
                             accum_sh.at[pl.ds(sid * RPT + t * K, K)], gsem1)
        if rem:
            pltpu.async_copy(rows1.at[pl.ds(0, rem)],
                             accum_sh.at[pl.ds(sid * RPT + full * K, rem)],
                             gsem1)
        for t in range(full):
            pltpu.make_async_copy(
                rows1, accum_sh.at[pl.ds(sid * RPT + t * K, K)], gsem1).wait()
        if rem:
            pltpu.make_async_copy(
                rows1.at[pl.ds(0, rem)],
                accum_sh.at[pl.ds(sid * RPT + full * K, rem)], gsem1).wait()
        gather(1, rows1, sb1, gsem1)
        plsc.subcore_barrier()

        def body(i, _):
            a = 2 * i
            gather_wait(rows0, gsem0)
            scat(a, rows0, db0)

            @pl.when(a + 2 < C)
            def _g0():
                gather(a + 2, rows0, sb0, gsem0)

            gather_wait(rows1, gsem1)
            scat(a + 1, rows1, db1)

            @pl.when(a + 3 < C)
            def _g1():
                gather(a + 3, rows1, sb1, gsem1)

            return _

        lax.fori_loop(0, C // 2, body, 0, unroll=False)
        if C % 2:
            gather_wait(rows0, gsem0)
            scat(C - 1, rows0, db0)
        plsc.subcore_barrier()

        pltpu.sync_copy(accum_sh.at[pl.ds(sid * RPT, RPT)],
                        out_hbm.at[cid, pl.ds(sid * RPT, RPT)])

    return agg(x, edge_index)


def _tc_mlp_body(eps_ref, x_ref, p_ref, w1_ref, b1_ref, g1_ref,
                 be1_ref, w2_ref, b2_ref, g2_ref, be2_ref, o_ref):
    n = x_ref.shape[0]
    h = (1.0 + eps_ref[0]) * x_ref[...] + (p_ref[0, :n] + p_ref[1, :n])
    h1 = jnp.dot(h, w1_ref[...], preferred_element_type=jnp.float32)
    h1 = h1 + b1_ref[...]
    m1 = jnp.mean(h1, axis=0, keepdims=True)
    v1 = jnp.mean((h1 - m1) ** 2, axis=0, keepdims=True)
    h1 = g1_ref[...] * (h1 - m1) * lax.rsqrt(v1 + _BN_EPS) + be1_ref[...]
    h1 = jnp.maximum(h1, 0.0)
    h2 = jnp.dot(h1, w2_ref[...], preferred_element_type=jnp.float32)
    h2 = h2 + b2_ref[...]
    m2 = jnp.mean(h2, axis=0, keepdims=True)
    v2 = jnp.mean((h2 - m2) ** 2, axis=0, keepdims=True)
    h2 = g2_ref[...] * (h2 - m2) * lax.rsqrt(v2 + _BN_EPS) + be2_ref[...]
    o_ref[...] = jnp.maximum(h2, 0.0)


def _tc_mlp(x, partials, W1t, b1, g1, be1, W2t, b2, g2, be2, eps):
    N, D = x.shape
    row = lambda a: a.reshape(1, D)
    return pl.pallas_call(
        _tc_mlp_body,
        out_shape=jax.ShapeDtypeStruct((N, D), jnp.float32),
        in_specs=[pl.BlockSpec(memory_space=pltpu.SMEM)]
        + [pl.BlockSpec(memory_space=pltpu.VMEM)] * 10,
    )(eps.reshape(1), x, partials, W1t, row(b1), row(g1), row(be1),
      W2t, row(b2), row(g2), row(be2))


def kernel(x, edge_index, batch, W1, b1, g1, be1, W2, b2, g2, be2, eps):
    N, D = x.shape
    E = edge_index.shape[1]
    info = plsc.get_sparse_core_info()
    NC, NS = info.num_cores, info.num_subcores
    NW = NC * NS
    EPW = E // NW
    assert E % NW == 0
    K = next(k for k in range(128, 0, -8) if EPW % k == 0)
    C = EPW // K
    assert C >= 2
    RPT = (((N + NS - 1) // NS) + 7) // 8 * 8
    Npad = RPT * NS

    partials = _sc_aggregate(x, edge_index,
                             NC=NC, NS=NS, C=C, K=K, RPT=RPT)
    return _tc_mlp(x, partials, W1.T, b1, g1, be1,
                   W2.T, b2, g2, be2, eps)
